# baseline (speedup 1.0000x reference)
import functools

import jax
import jax.numpy as jnp
from jax import lax
from jax.experimental import pallas as pl
from jax.experimental.pallas import tpu as pltpu
from jax.experimental.pallas import tpu_sc as plsc

BATCH = 4096
HIST = 50
EMBED = 32
CHUNK = 128


@functools.lru_cache(maxsize=None)
def _make_gather():
    info = plsc.get_sparse_core_info()
    nc, ns = info.num_cores, info.num_subcores
    nw = nc * ns
    total = BATCH * HIST
    b_per_w = total // nw
    n_chunks = b_per_w // CHUNK

    mesh = plsc.VectorSubcoreMesh(core_axis_name="c", subcore_axis_name="s")

    @functools.partial(
        pl.kernel,
        mesh=mesh,
        out_type=jax.ShapeDtypeStruct((total, EMBED), jnp.float32),
        scratch_types=[
            pltpu.VMEM((n_chunks, CHUNK), jnp.int32),
            pltpu.VMEM((CHUNK, EMBED), jnp.float32),
            pltpu.SemaphoreType.DMA,
        ],
        compiler_params=pltpu.CompilerParams(use_tc_tiling_on_sc=False),
    )
    def gather_kernel(idx_hbm, table_hbm, out_hbm, idx_v, rows_v, sem):
        wid = lax.axis_index("s") * nc + lax.axis_index("c")
        base = wid * b_per_w
        pltpu.sync_copy(idx_hbm.at[wid], idx_v)

        def body(j, carry):
            off = pl.multiple_of(base + j * CHUNK, CHUNK)
            pltpu.async_copy(table_hbm.at[idx_v.at[j]], rows_v, sem).wait()
            pltpu.sync_copy(rows_v, out_hbm.at[pl.ds(off, CHUNK)])
            return carry

        lax.fori_loop(0, n_chunks, body, 0)

    return gather_kernel, nw, n_chunks


def kernel(x, table):
    gather_fn, nw, n_chunks = _make_gather()
    idx = x.reshape(nw, n_chunks, CHUNK).astype(jnp.int32)
    out = gather_fn(idx, table)
    return out.reshape(BATCH, HIST, EMBED)


# speedup vs baseline: 1.0467x; 1.0467x over previous
import functools

import jax
import jax.numpy as jnp
from jax import lax
from jax.experimental import pallas as pl
from jax.experimental.pallas import tpu as pltpu
from jax.experimental.pallas import tpu_sc as plsc

BATCH = 4096
HIST = 50
EMBED = 32
CHUNK = 128


@functools.lru_cache(maxsize=None)
def _make_gather():
    info = plsc.get_sparse_core_info()
    nc, ns = info.num_cores, info.num_subcores
    nw = nc * ns
    total = BATCH * HIST
    b_per_w = total // nw
    n_chunks = b_per_w // CHUNK

    nbuf = 10
    assert n_chunks % nbuf == 0

    mesh = plsc.VectorSubcoreMesh(core_axis_name="c", subcore_axis_name="s")

    @functools.partial(
        pl.kernel,
        mesh=mesh,
        out_type=jax.ShapeDtypeStruct((total, EMBED), jnp.float32),
        scratch_types=[
            pltpu.VMEM((n_chunks, CHUNK), jnp.int32),
            pltpu.VMEM((nbuf, CHUNK, EMBED), jnp.float32),
            pltpu.SemaphoreType.DMA((nbuf,)),
            pltpu.SemaphoreType.DMA((nbuf,)),
        ],
        compiler_params=pltpu.CompilerParams(use_tc_tiling_on_sc=False),
    )
    def gather_kernel(idx_hbm, table_hbm, out_hbm, idx_v, rows, gsem, ssem):
        wid = lax.axis_index("s") * nc + lax.axis_index("c")
        base = wid * b_per_w
        pltpu.sync_copy(idx_hbm.at[wid], idx_v)

        def gather_start(c, b):
            pltpu.async_copy(table_hbm.at[idx_v.at[c]], rows.at[b], gsem.at[b])

        def gather_wait(c, b):
            pltpu.make_async_copy(
                table_hbm.at[idx_v.at[c]], rows.at[b], gsem.at[b]
            ).wait()

        def out_slot(c):
            off = pl.multiple_of(base + c * CHUNK, CHUNK)
            return out_hbm.at[pl.ds(off, CHUNK)]

        def store_start(c, b):
            pltpu.async_copy(rows.at[b], out_slot(c), ssem.at[b])

        def store_wait(c, b):
            pltpu.make_async_copy(rows.at[b], out_slot(c), ssem.at[b]).wait()

        for b in range(nbuf):
            gather_start(b, b)

        def outer(o, carry):
            for b in range(nbuf):
                c = o * nbuf + b
                gather_wait(c, b)
                store_start(c, b)
                store_wait(c, b)

                @pl.when(c + nbuf < n_chunks)
                def _():
                    gather_start(c + nbuf, b)

            return carry

        lax.fori_loop(0, n_chunks // nbuf, outer, 0)

    return gather_kernel, nw, n_chunks


def kernel(x, table):
    gather_fn, nw, n_chunks = _make_gather()
    idx = x.reshape(nw, n_chunks, CHUNK).astype(jnp.int32)
    out = gather_fn(idx, table)
    return out.reshape(BATCH, HIST, EMBED)
